# baseline (speedup 1.0000x reference)
import functools

import jax
import jax.numpy as jnp
from jax import lax
from jax.experimental import pallas as pl
from jax.experimental.pallas import tpu as pltpu
from jax.experimental.pallas import tpu_sc as plsc

VOCAB = 1_000_000
EMBED = 64
CTX = 200
V_BLK = 25_600


def _matvec_body(rows_ref, w_ref, b_ref, out_ref):
    avg = jnp.sum(rows_ref[...], axis=0, keepdims=True) * (1.0 / CTX)
    zero = jnp.zeros_like(avg)
    A = jnp.concatenate(
        [jnp.concatenate([avg, zero], axis=1),
         jnp.concatenate([zero, avg], axis=1)], axis=0)
    out_ref[...] = lax.dot_general(
        A, w_ref[...], (((1,), (1,)), ((), ())),
        preferred_element_type=jnp.float32) + b_ref[...]


def kernel(context_idx, embeddings, W, b):
    rows = lax.slice(embeddings, (0, 0), (CTX, EMBED))
    W2 = W.reshape(VOCAB // 2, 2 * EMBED)
    bT = b.reshape(VOCAB // 2, 2).T

    out = pl.pallas_call(
        _matvec_body,
        grid=(pl.cdiv(VOCAB // 2, V_BLK),),
        in_specs=[
            pl.BlockSpec((CTX, EMBED), lambda i: (0, 0)),
            pl.BlockSpec((V_BLK, 2 * EMBED), lambda i: (i, 0)),
            pl.BlockSpec((2, V_BLK), lambda i: (0, i)),
        ],
        out_specs=pl.BlockSpec((2, V_BLK), lambda i: (0, i)),
        out_shape=jax.ShapeDtypeStruct((2, VOCAB // 2), jnp.float32),
    )(rows, W2, bT)
    return out.T.reshape(1, VOCAB)


# speedup vs baseline: 2.8971x; 2.8971x over previous
import functools

import jax
import jax.numpy as jnp
from jax import lax
from jax.experimental import pallas as pl
from jax.experimental.pallas import tpu as pltpu
from jax.experimental.pallas import tpu_sc as plsc

VOCAB = 1_000_000
EMBED = 64
CTX = 200
V_BLK = 25_600


def _matvec_body(rows_ref, w_ref, b_ref, out_ref):
    avg = jnp.sum(rows_ref[...], axis=0, keepdims=True) * (1.0 / CTX)
    out_ref[...] = lax.dot_general(
        avg, w_ref[...], (((1,), (1,)), ((), ())),
        preferred_element_type=jnp.float32) + b_ref[...]


def kernel(context_idx, embeddings, W, b):
    rows = lax.slice(embeddings, (0, 0), (CTX, EMBED))

    logits = pl.pallas_call(
        _matvec_body,
        grid=(pl.cdiv(VOCAB, V_BLK),),
        in_specs=[
            pl.BlockSpec((CTX, EMBED), lambda i: (0, 0)),
            pl.BlockSpec((V_BLK, EMBED), lambda i: (i, 0)),
            pl.BlockSpec((1, V_BLK), lambda i: (0, i)),
        ],
        out_specs=pl.BlockSpec((1, V_BLK), lambda i: (0, i)),
        out_shape=jax.ShapeDtypeStruct((1, VOCAB), jnp.float32),
    )(rows, W, b.reshape(1, VOCAB))
    return logits


# speedup vs baseline: 6.8501x; 2.3645x over previous
import functools

import jax
import jax.numpy as jnp
from jax import lax
from jax.experimental import pallas as pl
from jax.experimental.pallas import tpu as pltpu
from jax.experimental.pallas import tpu_sc as plsc

VOCAB = 1_000_000
EMBED = 64
CTX = 200
HALF = VOCAB // 2
IDX_PAD = 224
K_BLK = 65_536
N_BLK = 65_536


def _sc_multihot_body(idx_hbm, ones_hbm, zeros_hbm, out_hbm,
                      idx_v, idxe_v, ones_v, spm, _sem):
    c = lax.axis_index("c")
    sid = lax.axis_index("s")

    @pl.when(sid == 0)
    def _():
        pltpu.sync_copy(zeros_hbm, spm.at[pl.ds(0, HALF)])
        pltpu.sync_copy(idx_hbm, idx_v)
        pltpu.sync_copy(ones_hbm, ones_v)
        base = c * HALF
        for j in range(2):
            for k in range(IDX_PAD // 32):
                sl = pl.ds(16 * k, 16)
                v = idx_v[j, sl]
                local = v - base
                valid = jnp.logical_and(local >= 0, local < HALF)
                idxe_v[j, sl] = jnp.where(valid, local, HALF)
        for j in range(2):
            pltpu.sync_copy(ones_v.at[j], spm.at[idxe_v.at[j]], add=True)
        pltpu.sync_copy(spm.at[pl.ds(0, HALF)], out_hbm.at[pl.ds(base, HALF)])


def _pool_body(s_ref, et_ref, out_ref):
    i = pl.program_id(0)
    sv = s_ref[...]
    limit = jnp.minimum(VOCAB - i * K_BLK, K_BLK)
    lane = lax.broadcasted_iota(jnp.int32, (1, K_BLK), 1)
    sv = jnp.where(lane < limit, sv, 0.0)
    z = lax.dot_general(sv, et_ref[...], (((1,), (1,)), ((), ())),
                        preferred_element_type=jnp.float32)

    @pl.when(i == 0)
    def _():
        out_ref[...] = z

    @pl.when(i > 0)
    def _():
        out_ref[...] += z


def _project_body(sum_ref, wt_ref, b_ref, out_ref):
    avg = sum_ref[...] * (1.0 / CTX)
    out_ref[...] = lax.dot_general(
        avg, wt_ref[...], (((1,), (0,)), ((), ())),
        preferred_element_type=jnp.float32) + b_ref[...]


def kernel(context_idx, embeddings, W, b):
    ci = context_idx.astype(jnp.int32)
    idx2 = jnp.pad(ci, (0, IDX_PAD - CTX),
                   constant_values=VOCAB).reshape(2, IDX_PAD // 2)
    ones2 = jnp.ones((2, IDX_PAD // 2), jnp.float32)
    zeros_half = jnp.zeros((HALF,), jnp.float32)

    mesh = plsc.VectorSubcoreMesh(core_axis_name="c", subcore_axis_name="s")
    multihot = pl.kernel(
        _sc_multihot_body,
        mesh=mesh,
        out_type=jax.ShapeDtypeStruct((VOCAB,), jnp.float32),
        scratch_types=[
            pltpu.VMEM((2, IDX_PAD // 2), jnp.int32),
            pltpu.VMEM((2, IDX_PAD // 2), jnp.int32),
            pltpu.VMEM((2, IDX_PAD // 2), jnp.float32),
            pltpu.VMEM_SHARED((HALF + 8,), jnp.float32),
            pltpu.SemaphoreType.DMA,
        ],
        compiler_params=pltpu.CompilerParams(use_tc_tiling_on_sc=False),
    )
    s = multihot(idx2, ones2, zeros_half)
    s2 = s.reshape(1, VOCAB)

    eT = embeddings.T
    wT = W.T

    sum_embed = pl.pallas_call(
        _pool_body,
        grid=(pl.cdiv(VOCAB, K_BLK),),
        in_specs=[
            pl.BlockSpec((1, K_BLK), lambda i: (0, i)),
            pl.BlockSpec((EMBED, K_BLK), lambda i: (0, i)),
        ],
        out_specs=pl.BlockSpec((1, EMBED), lambda i: (0, 0)),
        out_shape=jax.ShapeDtypeStruct((1, EMBED), jnp.float32),
    )(s2, eT)

    logits = pl.pallas_call(
        _project_body,
        grid=(pl.cdiv(VOCAB, N_BLK),),
        in_specs=[
            pl.BlockSpec((1, EMBED), lambda i: (0, 0)),
            pl.BlockSpec((EMBED, N_BLK), lambda i: (0, i)),
            pl.BlockSpec((1, N_BLK), lambda i: (0, i)),
        ],
        out_specs=pl.BlockSpec((1, N_BLK), lambda i: (0, i)),
        out_shape=jax.ShapeDtypeStruct((1, VOCAB), jnp.float32),
    )(sum_embed, wT, b.reshape(1, VOCAB))
    return logits


# speedup vs baseline: 7.9510x; 1.1607x over previous
import functools

import jax
import jax.numpy as jnp
from jax import lax
from jax.experimental import pallas as pl
from jax.experimental.pallas import tpu as pltpu
from jax.experimental.pallas import tpu_sc as plsc

VOCAB = 1_000_000
EMBED = 64
CTX = 200
HALF = VOCAB // 2
IDX_PAD = 224
K_BLK = 65_536
N_BLK = 65_536


def _sc_multihot_body(idx_hbm, ones_hbm, zeros_hbm, out_hbm,
                      idx_v, idxe_v, ones_v, spm, _sem):
    c = lax.axis_index("c")
    sid = lax.axis_index("s")

    @pl.when(sid == 0)
    def _():
        pltpu.sync_copy(zeros_hbm, spm.at[pl.ds(0, HALF)])
        pltpu.sync_copy(idx_hbm, idx_v)
        pltpu.sync_copy(ones_hbm, ones_v)
        base = c * HALF
        for j in range(2):
            for k in range(IDX_PAD // 32):
                sl = pl.ds(16 * k, 16)
                v = idx_v[j, sl]
                local = v - base
                valid = jnp.logical_and(local >= 0, local < HALF)
                idxe_v[j, sl] = jnp.where(valid, local, HALF)
        for j in range(2):
            pltpu.sync_copy(ones_v.at[j], spm.at[idxe_v.at[j]], add=True)
        pltpu.sync_copy(spm.at[pl.ds(0, HALF)], out_hbm.at[pl.ds(base, HALF)])


def _pool_body(s_ref, et_ref, out_ref):
    i = pl.program_id(0)
    sv = s_ref[...].reshape(1, K_BLK)
    limit = jnp.minimum(VOCAB - i * K_BLK, K_BLK)
    lane = lax.broadcasted_iota(jnp.int32, (1, K_BLK), 1)
    sv = jnp.where(lane < limit, sv, 0.0)
    z = lax.dot_general(sv, et_ref[...], (((1,), (1,)), ((), ())),
                        preferred_element_type=jnp.float32)

    @pl.when(i == 0)
    def _():
        out_ref[...] = z

    @pl.when(i > 0)
    def _():
        out_ref[...] += z


def _project_body(sum_ref, wt_ref, b_ref, out_ref):
    avg = sum_ref[...] * (1.0 / CTX)
    out_ref[...] = lax.dot_general(
        avg, wt_ref[...], (((1,), (0,)), ((), ())),
        preferred_element_type=jnp.float32) + b_ref[...].reshape(1, N_BLK)


def kernel(context_idx, embeddings, W, b):
    ci = context_idx.astype(jnp.int32)
    idx2 = jnp.pad(ci, (0, IDX_PAD - CTX),
                   constant_values=VOCAB).reshape(2, IDX_PAD // 2)
    ones2 = jnp.ones((2, IDX_PAD // 2), jnp.float32)
    zeros_half = jnp.zeros((HALF,), jnp.float32)

    mesh = plsc.VectorSubcoreMesh(core_axis_name="c", subcore_axis_name="s")
    multihot = pl.kernel(
        _sc_multihot_body,
        mesh=mesh,
        out_type=jax.ShapeDtypeStruct((VOCAB,), jnp.float32),
        scratch_types=[
            pltpu.VMEM((2, IDX_PAD // 2), jnp.int32),
            pltpu.VMEM((2, IDX_PAD // 2), jnp.int32),
            pltpu.VMEM((2, IDX_PAD // 2), jnp.float32),
            pltpu.VMEM_SHARED((HALF + 8,), jnp.float32),
            pltpu.SemaphoreType.DMA,
        ],
        compiler_params=pltpu.CompilerParams(use_tc_tiling_on_sc=False),
    )
    s = multihot(idx2, ones2, zeros_half)

    eT = embeddings.T
    wT = W.T

    sum_embed = pl.pallas_call(
        _pool_body,
        grid=(pl.cdiv(VOCAB, K_BLK),),
        in_specs=[
            pl.BlockSpec((K_BLK,), lambda i: (i,)),
            pl.BlockSpec((EMBED, K_BLK), lambda i: (0, i)),
        ],
        out_specs=pl.BlockSpec((1, EMBED), lambda i: (0, 0)),
        out_shape=jax.ShapeDtypeStruct((1, EMBED), jnp.float32),
    )(s, eT)

    logits = pl.pallas_call(
        _project_body,
        grid=(pl.cdiv(VOCAB, N_BLK),),
        in_specs=[
            pl.BlockSpec((1, EMBED), lambda i: (0, 0)),
            pl.BlockSpec((EMBED, N_BLK), lambda i: (0, i)),
            pl.BlockSpec((N_BLK,), lambda i: (i,)),
        ],
        out_specs=pl.BlockSpec((1, N_BLK), lambda i: (0, i)),
        out_shape=jax.ShapeDtypeStruct((1, VOCAB), jnp.float32),
    )(sum_embed, wT, b)
    return logits
